# Initial kernel scaffold; baseline (speedup 1.0000x reference)
import jax
import jax.numpy as jnp
from jax.experimental import pallas as pl


def kernel(input_sequences, target_sequences, scale_factors, W1, b1, W2, b2, W_ih, W_hh, b_ih, b_hh):
    raise NotImplementedError("write your pallas kernel here")


# speedup vs baseline: 69.2342x; 69.2342x over previous
import jax
import jax.numpy as jnp
from jax.experimental import pallas as pl
from jax.experimental.pallas import tpu as pltpu

F32 = jnp.float32
N, T = 8, 16
NT = N * T
HID = 128
DT = 0.04


def _rollout_kernel(frames_ref, ch_ref, qi_ref, pos_ref, qt_ref, scale_ref,
                    invs_ref, w1_ref, b1_ref, w2_ref, b2_ref, wih_ref,
                    bs_ref, whh_ref, out_ref, cs_ref):
    frames = frames_ref[...].astype(jnp.int32)
    q = jnp.concatenate([qi_ref[...], qt_ref[...]], axis=0)
    mask = (q.astype(jnp.int32) == frames).astype(F32)
    cnt = jnp.sum(mask, axis=1, keepdims=True)
    s = jnp.dot(mask, ch_ref[...], preferred_element_type=F32)

    pos = pos_ref[...]
    pooled0 = pos * cnt[:NT] - s[:NT]
    iseq0 = jnp.concatenate([pos, pooled0], axis=1)

    cs_ref[:, 0:1] = cnt[NT:]
    cs_ref[:, 1:5] = s[NT:]
    cs_ref[:, 5:8] = jnp.zeros((NT, 3), F32)

    w1 = w1_ref[...]
    b1 = b1_ref[...]
    w2 = w2_ref[...]
    b2 = b2_ref[...]
    wih = wih_ref[...]
    bs = bs_ref[...]
    whh = whh_ref[...]
    scale = scale_ref[...]
    invs = invs_ref[...]

    def step(i, iseq):
        fx = jnp.tanh(jnp.dot(iseq, w1, preferred_element_type=F32) + b1)
        fx = jnp.dot(fx, w2, preferred_element_type=F32) + b2
        x_all = jnp.dot(fx, wih, preferred_element_type=F32) + bs

        h = jnp.zeros((N, HID), F32)
        c = jnp.zeros((N, HID), F32)
        for t in range(T):
            g = x_all[t * N:(t + 1) * N, :] + jnp.dot(
                h, whh, preferred_element_type=F32)
            ig = jax.nn.sigmoid(g[:, 0:128])
            fg = jax.nn.sigmoid(g[:, 128:256])
            gg = jnp.tanh(g[:, 256:384])
            og = jax.nn.sigmoid(g[:, 384:512])
            c = fg * c + ig * gg
            h = og * jnp.tanh(c)

        iseq = iseq * scale
        last = iseq[NT - N:NT, :]
        xv = last[:, 2:3] + DT * h[:, 0:1]
        yv = last[:, 3:4] + DT * h[:, 1:2]
        x = last[:, 0:1] + DT * xv
        y = last[:, 1:2] + DT * yv
        u = jnp.concatenate([x, y, xv, yv], axis=1) * invs
        out_ref[pl.ds(i * N, N), :] = u

        cs = cs_ref[pl.ds(i * N, N), :]
        nr = jnp.concatenate([u, cs[:, 0:1] * u - cs[:, 1:5]], axis=1)
        return jnp.concatenate([iseq[N:], nr], axis=0)

    jax.lax.fori_loop(0, T, step, iseq0)


@jax.jit
def kernel(input_sequences, target_sequences, scale_factors, W1, b1, W2, b2,
           W_ih, W_hh, b_ih, b_hh):
    frames_row = input_sequences[:, :, 0].reshape(1, NT)
    ch = input_sequences[:, :, 2:6].reshape(NT, 4)
    q_init = input_sequences[:, :, 0].T.reshape(NT, 1)
    pos_init = jnp.transpose(input_sequences[:, :, 2:6], (1, 0, 2)).reshape(NT, 4)
    q_tgt = target_sequences[:, :, 0].T.reshape(NT, 1)
    scale_row = jnp.concatenate([scale_factors, jnp.ones((4,), F32)]).reshape(1, 8)
    inv_scale = (1.0 / scale_factors).reshape(1, 4)

    out = pl.pallas_call(
        _rollout_kernel,
        out_shape=jax.ShapeDtypeStruct((NT, 4), F32),
        scratch_shapes=[pltpu.VMEM((NT, 8), F32)],
    )(frames_row, ch, q_init, pos_init, q_tgt, scale_row, inv_scale,
      W1.T, b1.reshape(1, HID), W2.T, b2.reshape(1, HID),
      W_ih.T, (b_ih + b_hh).reshape(1, 4 * HID), W_hh.T)

    return out.reshape(T, N, 4).transpose(1, 0, 2)


# speedup vs baseline: 116.7880x; 1.6869x over previous
import numpy as np

import jax
import jax.numpy as jnp
from jax.experimental import pallas as pl

F32 = jnp.float32
N, T = 8, 16
NT = N * T
HID = 128
DT = 0.04
HP = jax.lax.Precision.HIGHEST


def _rollout_kernel(frames_ref, ch_ref, qi_ref, pos_ref, qt_ref, scale_ref,
                    invs8_ref, m_ref, e_ref, c_ref, w1_ref, b1_ref,
                    w2ih_ref, bs_ref, whh_ref, out_ref):
    frames = frames_ref[...].astype(jnp.int32)
    q = jnp.concatenate([qi_ref[...], qt_ref[...]], axis=0)
    mask = (q.astype(jnp.int32) == frames).astype(F32)
    cnt = jnp.sum(mask, axis=1, keepdims=True)
    s = jnp.dot(mask, ch_ref[...], preferred_element_type=F32)

    pos = pos_ref[...]
    pooled0 = pos * cnt[:NT] - s[:NT]
    iseq0 = jnp.concatenate([pos, pooled0], axis=1)

    zeros4 = jnp.zeros((NT, 4), F32)
    cntpat = jnp.concatenate([zeros4 + 1.0, zeros4 + cnt[NT:]], axis=1)
    spat = jnp.concatenate([zeros4, s[NT:]], axis=1)

    w1 = w1_ref[...]
    b1 = b1_ref[...]
    w2ih = w2ih_ref[...]
    bs = bs_ref[...]
    whh = whh_ref[...]
    scale = scale_ref[...]
    invs8 = invs8_ref[...]
    mmat = m_ref[...]
    emat = e_ref[...]
    cmat = c_ref[...]

    def mlp(rows):
        fx = jnp.tanh(jnp.dot(rows, w1, preferred_element_type=F32) + b1)
        return jnp.dot(fx, w2ih, preferred_element_type=F32) + bs

    spow = [jnp.ones((1, 8), F32)]
    for k in range(1, T):
        spow.append(spow[-1] * scale)

    X = {}
    for i in range(T):
        xi = mlp(iseq0[i * N:] * spow[i] if i else iseq0)
        for t in range(T - i):
            X[(i, t)] = xi[t * N:(t + 1) * N]

    H = [None] * T
    C = [None] * T
    P = [None] * T
    for sl in range(3 * (T - 1) + T):
        for i in range(T):
            t = sl - 3 * i
            if t < 0 or t >= T:
                continue
            xt = X[(i, t)]
            g = xt if t == 0 else xt + jnp.dot(
                H[i], whh, preferred_element_type=F32)
            tg = jnp.tanh(g)
            ti = tg[:, 0:128]
            tf = tg[:, 128:256]
            tc = tg[:, 256:384]
            to = tg[:, 384:512]
            cc = (0.5 * ((ti + 1.0) * tc) if t == 0
                  else 0.5 * ((tf + 1.0) * C[i] + (ti + 1.0) * tc))
            hh = (0.5 * (to + 1.0)) * jnp.tanh(cc)
            H[i], C[i] = hh, cc

            if t == T - 1:
                last = (iseq0[NT - N:NT] if i == 0 else P[i - 1]) * scale
                u = (jnp.dot(last, mmat, preferred_element_type=F32,
                             precision=HP)
                     + jnp.dot(hh, emat, preferred_element_type=F32,
                               precision=HP)) * invs8
                out_ref[i * N:(i + 1) * N, :] = u
                if i < T - 1:
                    nr = (jnp.dot(u, cmat, preferred_element_type=F32,
                                  precision=HP)
                          * cntpat[i * N:(i + 1) * N]
                          - spat[i * N:(i + 1) * N])
                    P[i] = nr
                    nlad = T - 1 - i
                    lad = jnp.concatenate(
                        [nr * spow[k] for k in range(nlad)], axis=0)
                    xl = mlp(lad)
                    for k in range(nlad):
                        X[(i + 1 + k, T - 1 - k)] = xl[k * N:(k + 1) * N]


@jax.jit
def kernel(input_sequences, target_sequences, scale_factors, W1, b1, W2, b2,
           W_ih, W_hh, b_ih, b_hh):
    frames_row = input_sequences[:, :, 0].reshape(1, NT)
    ch = input_sequences[:, :, 2:6].reshape(NT, 4)
    q_init = input_sequences[:, :, 0].T.reshape(NT, 1)
    pos_init = jnp.transpose(input_sequences[:, :, 2:6], (1, 0, 2)).reshape(NT, 4)
    q_tgt = target_sequences[:, :, 0].T.reshape(NT, 1)
    scale_row = jnp.concatenate([scale_factors, jnp.ones((4,), F32)]).reshape(1, 8)
    inv_scale = (1.0 / scale_factors).reshape(1, 4)

    mnp = np.zeros((8, 4), np.float32)
    mnp[0, 0] = 1.0
    mnp[1, 1] = 1.0
    mnp[2, 0] = DT
    mnp[2, 2] = 1.0
    mnp[3, 1] = DT
    mnp[3, 3] = 1.0
    enp = np.zeros((HID, 4), np.float32)
    enp[0, 0] = DT * DT
    enp[0, 2] = DT
    enp[1, 1] = DT * DT
    enp[1, 3] = DT
    cnp = np.concatenate([np.eye(4, dtype=np.float32)] * 2, axis=1)

    col_scale = np.ones((1, 4 * HID), np.float32) * 0.5
    col_scale[0, 2 * HID:3 * HID] = 1.0
    hp = jax.lax.Precision.HIGHEST
    w2ih = jnp.dot(W2.T, W_ih.T, precision=hp) * col_scale
    bsum = (jnp.dot(b2, W_ih.T, precision=hp)
            + b_ih + b_hh).reshape(1, 4 * HID) * col_scale
    whht = W_hh.T * col_scale

    out = pl.pallas_call(
        _rollout_kernel,
        out_shape=jax.ShapeDtypeStruct((NT, 4), F32),
    )(frames_row, ch, q_init, pos_init, q_tgt, scale_row, inv_scale,
      jnp.asarray(mnp), jnp.asarray(enp), jnp.asarray(cnp),
      W1.T, b1.reshape(1, HID), w2ih, bsum, whht)

    return out.reshape(T, N, 4).transpose(1, 0, 2)
